# baseline (speedup 1.0000x reference)
import functools

import jax
import jax.numpy as jnp
from jax import lax
from jax.experimental import pallas as pl

_B, _C, _P = 1024, 100000, 50
_PPAD = 64
_R = 32
_NX = 8
_CB = 12544
_Q = _CB // 128
_CLAST = _C - (_NX - 1) * _CB


def _loss_body(*refs):
    x_refs = refs[:_NX]
    tgrp_ref, tmod_ref, o_ref = refs[_NX], refs[_NX + 1], refs[_NX + 2]

    def masked(j, x):
        if j < _NX - 1:
            return x
        tail = lax.broadcasted_iota(jnp.int32, (_R, _CB), 1) < _CLAST
        return jnp.where(tail, x, -jnp.inf)

    m = jnp.full((_R, 1), 0.0, dtype=jnp.float32)
    for j, xr in enumerate(x_refs):
        m = jnp.maximum(m, jnp.max(masked(j, xr[...]), axis=1, keepdims=True))
    s = jnp.exp(-m)
    for j, xr in enumerate(x_refs):
        s = s + jnp.sum(jnp.exp(masked(j, xr[...]) - m), axis=1, keepdims=True)
    all_loss = m + jnp.log(s)

    tgrp = tgrp_ref[...]
    q_iota = lax.broadcasted_iota(jnp.int32, (_R, _PPAD, _NX * _Q), 2)
    sel = (tgrp[..., None] == q_iota).astype(jnp.bfloat16)
    parts = []
    for j, xr in enumerate(x_refs):
        xv = xr[...]
        if j == _NX - 1:
            tail = lax.broadcasted_iota(jnp.int32, (_R, _CB), 1) < _CLAST
            xv = jnp.where(tail, xv, 0.0)
        parts.append(xv.astype(jnp.bfloat16))
    x3 = jnp.concatenate(parts, axis=1).reshape(_R, _NX * _Q, 128)
    z = lax.dot_general(
        sel, x3, (((2,), (1,)), ((0,), (0,))),
        preferred_element_type=jnp.float32,
    )
    lane = lax.broadcasted_iota(jnp.int32, (_R, _PPAD, 128), 2)
    tmod = tmod_ref[...]
    g = jnp.sum(jnp.where(lane == tmod[..., None], z, 0.0), axis=2)

    valid = lax.broadcasted_iota(jnp.int32, (_R, _PPAD), 1) < _P
    gmask = jnp.where(valid, g, -jnp.inf)
    m_p = jnp.max(gmask, axis=1, keepdims=True)
    s_p = jnp.sum(jnp.where(valid, jnp.exp(g - m_p), 0.0), axis=1, keepdims=True)
    lse_pos = m_p + jnp.log(s_p)

    zneg = jnp.where(valid, -g, -jnp.inf)
    m_n = jnp.maximum(jnp.max(zneg, axis=1, keepdims=True), 0.0)
    s_n = jnp.sum(jnp.where(valid, jnp.exp(-g - m_n), 0.0), axis=1, keepdims=True)
    pos_loss = m_n + jnp.log(s_n + jnp.exp(-m_n))

    aux = jnp.clip(1.0 - jnp.exp(lse_pos - all_loss), 1e-12, 1.0)
    o_ref[...] = pos_loss + all_loss + jnp.log(aux)


_loss_call = pl.pallas_call(
    _loss_body,
    grid=(_B // _R,),
    in_specs=[
        pl.BlockSpec((_R, _CB), functools.partial(lambda j, i: (i, j), j))
        for j in range(_NX)
    ]
    + [
        pl.BlockSpec((_R, _PPAD), lambda i: (i, 0)),
        pl.BlockSpec((_R, _PPAD), lambda i: (i, 0)),
    ],
    out_specs=pl.BlockSpec((_R, 1), lambda i: (i, 0)),
    out_shape=jax.ShapeDtypeStruct((_B, 1), jnp.float32),
)


def kernel(input, target):
    tgt = jnp.concatenate([target, target[:, : _PPAD - _P]], axis=1)
    out = _loss_call(*([input] * _NX), tgt // 128, tgt % 128)
    return out.reshape(_B)


# speedup vs baseline: 1.0327x; 1.0327x over previous
import functools

import jax
import jax.numpy as jnp
from jax import lax
from jax.experimental import pallas as pl

_B, _C, _P = 1024, 100000, 50
_PPAD = 64
_R = 16
_NX = 8
_CB = 12544
_Q = _CB // 128
_CLAST = _C - (_NX - 1) * _CB


def _loss_body(*refs):
    x_refs = refs[:_NX]
    tgrp_ref, tmod_ref, o_ref = refs[_NX], refs[_NX + 1], refs[_NX + 2]

    def masked(j, x):
        if j < _NX - 1:
            return x
        tail = lax.broadcasted_iota(jnp.int32, (_R, _CB), 1) < _CLAST
        return jnp.where(tail, x, -jnp.inf)

    m = jnp.full((_R, 1), 0.0, dtype=jnp.float32)
    for j, xr in enumerate(x_refs):
        m = jnp.maximum(m, jnp.max(masked(j, xr[...]), axis=1, keepdims=True))
    s = jnp.exp(-m)
    for j, xr in enumerate(x_refs):
        s = s + jnp.sum(jnp.exp(masked(j, xr[...]) - m), axis=1, keepdims=True)
    all_loss = m + jnp.log(s)

    tgrp = tgrp_ref[...]
    q_iota = lax.broadcasted_iota(jnp.int32, (_R, _PPAD, _NX * _Q), 2)
    sel = (tgrp[..., None] == q_iota).astype(jnp.bfloat16)
    parts = []
    for j, xr in enumerate(x_refs):
        xv = xr[...]
        if j == _NX - 1:
            tail = lax.broadcasted_iota(jnp.int32, (_R, _CB), 1) < _CLAST
            xv = jnp.where(tail, xv, 0.0)
        parts.append(xv.astype(jnp.bfloat16))
    x3 = jnp.concatenate(parts, axis=1).reshape(_R, _NX * _Q, 128)
    z = lax.dot_general(
        sel, x3, (((2,), (1,)), ((0,), (0,))),
        preferred_element_type=jnp.float32,
    )
    lane = lax.broadcasted_iota(jnp.int32, (_R, _PPAD, 128), 2)
    tmod = tmod_ref[...]
    g = jnp.sum(jnp.where(lane == tmod[..., None], z, 0.0), axis=2)

    valid = lax.broadcasted_iota(jnp.int32, (_R, _PPAD), 1) < _P
    gmask = jnp.where(valid, g, -jnp.inf)
    m_p = jnp.max(gmask, axis=1, keepdims=True)
    s_p = jnp.sum(jnp.where(valid, jnp.exp(g - m_p), 0.0), axis=1, keepdims=True)
    lse_pos = m_p + jnp.log(s_p)

    zneg = jnp.where(valid, -g, -jnp.inf)
    m_n = jnp.maximum(jnp.max(zneg, axis=1, keepdims=True), 0.0)
    s_n = jnp.sum(jnp.where(valid, jnp.exp(-g - m_n), 0.0), axis=1, keepdims=True)
    pos_loss = m_n + jnp.log(s_n + jnp.exp(-m_n))

    aux = jnp.clip(1.0 - jnp.exp(lse_pos - all_loss), 1e-12, 1.0)
    o_ref[...] = pos_loss + all_loss + jnp.log(aux)


_loss_call = pl.pallas_call(
    _loss_body,
    grid=(_B // _R,),
    in_specs=[
        pl.BlockSpec((_R, _CB), functools.partial(lambda j, i: (i, j), j))
        for j in range(_NX)
    ]
    + [
        pl.BlockSpec((_R, _PPAD), lambda i: (i, 0)),
        pl.BlockSpec((_R, _PPAD), lambda i: (i, 0)),
    ],
    out_specs=pl.BlockSpec((_R, 1), lambda i: (i, 0)),
    out_shape=jax.ShapeDtypeStruct((_B, 1), jnp.float32),
)


def kernel(input, target):
    tgt = jnp.concatenate([target, target[:, : _PPAD - _P]], axis=1)
    out = _loss_call(*([input] * _NX), tgt // 128, tgt % 128)
    return out.reshape(_B)
